# baseline (speedup 1.0000x reference)
import functools

import jax
import jax.numpy as jnp
from jax import lax
from jax.experimental import pallas as pl
from jax.experimental.pallas import tpu as pltpu
from jax.experimental.pallas import tpu_sc as plsc

BATCH = 4096
HIST = 50
EMBED_DIM = 64
PLANE_H = 56
PLANE_W = 128
NUM_WORKERS = 32
BROWS_PER_W = BATCH // NUM_WORKERS
NBUF = 2

_mesh = plsc.VectorSubcoreMesh(core_axis_name="c", subcore_axis_name="s")


@functools.partial(
    pl.kernel,
    mesh=_mesh,
    out_type=jax.ShapeDtypeStruct((BATCH, PLANE_H, PLANE_W), jnp.float32),
    compiler_params=pltpu.CompilerParams(use_tc_tiling_on_sc=False),
    scratch_types=[
        pltpu.VMEM((HIST, BROWS_PER_W), jnp.int32),
        pltpu.VMEM((NBUF, BROWS_PER_W, EMBED_DIM), jnp.float32),
        pltpu.SemaphoreType.DMA((NBUF,)),
        pltpu.SemaphoreType.DMA((NBUF,)),
    ],
)
def _gather_kernel(table_hbm, idx_hbm, out_hbm, idx_v, stage, gsems, osems):
    wid = lax.axis_index("s") * 2 + lax.axis_index("c")
    brow0 = wid * BROWS_PER_W
    pltpu.sync_copy(idx_hbm.at[:, pl.ds(brow0, BROWS_PER_W)], idx_v)

    def _wait_out(b):
        pltpu.make_async_copy(
            stage.at[b],
            out_hbm.at[pl.ds(brow0, BROWS_PER_W), 0, pl.ds(0, EMBED_DIM)],
            osems.at[b]).wait()

    def group(g, carry):
        gh = [None] * NBUF
        for b in range(NBUF):
            h = NBUF * g + b

            @pl.when(g > 0)
            def _(b=b):
                _wait_out(b)

            gh[b] = pltpu.async_copy(
                table_hbm.at[idx_v.at[h]],
                stage.at[b],
                gsems.at[b])
        for b in range(NBUF):
            h = NBUF * g + b
            gh[b].wait()
            pltpu.async_copy(
                stage.at[b],
                out_hbm.at[pl.ds(brow0, BROWS_PER_W), h, pl.ds(0, EMBED_DIM)],
                osems.at[b])
        return carry

    lax.fori_loop(0, HIST // NBUF, group, 0, unroll=False)
    for b in range(NBUF):
        _wait_out(b)


def kernel(x, text_embedding_vectors):
    y = _gather_kernel(text_embedding_vectors, x.T)
    return y[:, :HIST, :EMBED_DIM]


# speedup vs baseline: 1.0455x; 1.0455x over previous
import functools

import jax
import jax.numpy as jnp
from jax import lax
from jax.experimental import pallas as pl
from jax.experimental.pallas import tpu as pltpu
from jax.experimental.pallas import tpu_sc as plsc

BATCH = 4096
HIST = 50
EMBED_DIM = 64
PLANE_H = 56
PLANE_W = 128
NUM_WORKERS = 32
BROWS_PER_W = BATCH // NUM_WORKERS
PER_WORKER = BROWS_PER_W * HIST
BCHUNK = 8
NUM_CHUNKS = BROWS_PER_W // BCHUNK
NBUF = 4

_mesh = plsc.VectorSubcoreMesh(core_axis_name="c", subcore_axis_name="s")


@functools.partial(
    pl.kernel,
    mesh=_mesh,
    out_type=jax.ShapeDtypeStruct((BATCH, PLANE_H, PLANE_W), jnp.float32),
    compiler_params=pltpu.CompilerParams(use_tc_tiling_on_sc=False),
    scratch_types=[
        pltpu.VMEM((PER_WORKER,), jnp.int32),
        pltpu.VMEM((NBUF, BCHUNK * HIST, EMBED_DIM), jnp.float32),
        pltpu.SemaphoreType.DMA((NBUF,)),
        pltpu.SemaphoreType.DMA((NBUF,)),
    ],
)
def _gather_kernel(table_hbm, idx_hbm, out_hbm, idx_v, stage, gsems, osems):
    wid = lax.axis_index("s") * 2 + lax.axis_index("c")
    pltpu.sync_copy(idx_hbm.at[pl.ds(wid * PER_WORKER, PER_WORKER)], idx_v)
    brow0 = wid * BROWS_PER_W

    def _wait_outs(b):
        for r in range(BCHUNK):
            pltpu.make_async_copy(
                stage.at[b, pl.ds(r * HIST, HIST)],
                out_hbm.at[brow0, pl.ds(0, HIST), pl.ds(0, EMBED_DIM)],
                osems.at[b]).wait()

    def group(g, carry):
        gh = [None] * NBUF
        for b in range(NBUF):
            j = NBUF * g + b

            @pl.when(g > 0)
            def _(b=b):
                _wait_outs(b)

            gh[b] = pltpu.async_copy(
                table_hbm.at[idx_v.at[pl.ds(j * BCHUNK * HIST, BCHUNK * HIST)]],
                stage.at[b],
                gsems.at[b])
        for b in range(NBUF):
            j = NBUF * g + b
            gh[b].wait()
            for r in range(BCHUNK):
                pltpu.async_copy(
                    stage.at[b, pl.ds(r * HIST, HIST)],
                    out_hbm.at[brow0 + j * BCHUNK + r,
                               pl.ds(0, HIST), pl.ds(0, EMBED_DIM)],
                    osems.at[b])
        return carry

    lax.fori_loop(0, NUM_CHUNKS // NBUF, group, 0, unroll=False)
    for b in range(NBUF):
        _wait_outs(b)


def kernel(x, text_embedding_vectors):
    y = _gather_kernel(text_embedding_vectors, jnp.clip(x.reshape(-1), 0, 99999))
    return y[:, :HIST, :EMBED_DIM]


# speedup vs baseline: 1.0479x; 1.0023x over previous
import functools

import jax
import jax.numpy as jnp
from jax import lax
from jax.experimental import pallas as pl
from jax.experimental.pallas import tpu as pltpu
from jax.experimental.pallas import tpu_sc as plsc

BATCH = 4096
HIST = 50
EMBED_DIM = 64
PLANE_H = 56
PLANE_W = 128
NUM_WORKERS = 32
BROWS_PER_W = BATCH // NUM_WORKERS
PER_WORKER = BROWS_PER_W * HIST
BCHUNK = 8
NUM_CHUNKS = BROWS_PER_W // BCHUNK
NBUF = 4

_mesh = plsc.VectorSubcoreMesh(core_axis_name="c", subcore_axis_name="s")


@functools.partial(
    pl.kernel,
    mesh=_mesh,
    out_type=jax.ShapeDtypeStruct((BATCH, PLANE_H, PLANE_W), jnp.float32),
    compiler_params=pltpu.CompilerParams(use_tc_tiling_on_sc=False),
    scratch_types=[
        pltpu.VMEM((PER_WORKER,), jnp.int32),
        pltpu.VMEM((NBUF, BCHUNK * HIST, EMBED_DIM), jnp.float32),
        pltpu.SemaphoreType.DMA((NBUF,)),
        pltpu.SemaphoreType.DMA((NBUF,)),
    ],
)
def _gather_kernel(table_hbm, idx_hbm, out_hbm, idx_v, stage, gsems, osems):
    wid = lax.axis_index("s") * 2 + lax.axis_index("c")
    pltpu.sync_copy(idx_hbm.at[pl.ds(wid * PER_WORKER, PER_WORKER)], idx_v)
    brow0 = wid * BROWS_PER_W

    def _wait_outs(b):
        for r in range(BCHUNK):
            pltpu.make_async_copy(
                stage.at[b, pl.ds(r * HIST, HIST)],
                out_hbm.at[brow0, pl.ds(0, HIST), pl.ds(0, EMBED_DIM)],
                osems.at[b]).wait()

    def group(g, carry):
        gh = [None] * NBUF
        for b in range(NBUF):
            j = NBUF * g + b

            @pl.when(g > 0)
            def _(b=b):
                _wait_outs(b)

            gh[b] = pltpu.async_copy(
                table_hbm.at[idx_v.at[pl.ds(j * BCHUNK * HIST, BCHUNK * HIST)]],
                stage.at[b],
                gsems.at[b])
        for b in range(NBUF):
            j = NBUF * g + b
            gh[b].wait()
            for r in range(BCHUNK):
                pltpu.async_copy(
                    stage.at[b, pl.ds(r * HIST, HIST)],
                    out_hbm.at[brow0 + j * BCHUNK + r,
                               pl.ds(0, HIST), pl.ds(0, EMBED_DIM)],
                    osems.at[b])
        return carry

    lax.fori_loop(0, NUM_CHUNKS // NBUF, group, 0, unroll=False)
    for b in range(NBUF):
        _wait_outs(b)


def kernel(x, text_embedding_vectors):
    y = _gather_kernel(text_embedding_vectors, x.reshape(-1))
    return y[:, :HIST, :EMBED_DIM]


# speedup vs baseline: 1.0539x; 1.0057x over previous
import functools

import jax
import jax.numpy as jnp
from jax import lax
from jax.experimental import pallas as pl
from jax.experimental.pallas import tpu as pltpu
from jax.experimental.pallas import tpu_sc as plsc

BATCH = 4096
HIST = 50
EMBED_DIM = 64
PLANE_H = 56
PLANE_W = 128
NUM_WORKERS = 32
BROWS_PER_W = BATCH // NUM_WORKERS
PER_WORKER = BROWS_PER_W * HIST
BCHUNK = 16
NUM_CHUNKS = BROWS_PER_W // BCHUNK
NBUF = 2

_mesh = plsc.VectorSubcoreMesh(core_axis_name="c", subcore_axis_name="s")


@functools.partial(
    pl.kernel,
    mesh=_mesh,
    out_type=jax.ShapeDtypeStruct((BATCH, PLANE_H, PLANE_W), jnp.float32),
    compiler_params=pltpu.CompilerParams(use_tc_tiling_on_sc=False),
    scratch_types=[
        pltpu.VMEM((PER_WORKER,), jnp.int32),
        pltpu.VMEM((NBUF, BCHUNK * HIST, EMBED_DIM), jnp.float32),
        pltpu.SemaphoreType.DMA((NBUF,)),
        pltpu.SemaphoreType.DMA((NBUF,)),
    ],
)
def _gather_kernel(table_hbm, idx_hbm, out_hbm, idx_v, stage, gsems, osems):
    wid = lax.axis_index("s") * 2 + lax.axis_index("c")
    pltpu.sync_copy(idx_hbm.at[pl.ds(wid * PER_WORKER, PER_WORKER)], idx_v)
    brow0 = wid * BROWS_PER_W

    def _wait_outs(b):
        for r in range(BCHUNK):
            pltpu.make_async_copy(
                stage.at[b, pl.ds(r * HIST, HIST)],
                out_hbm.at[brow0, pl.ds(0, HIST), pl.ds(0, EMBED_DIM)],
                osems.at[b]).wait()

    def group(g, carry):
        gh = [None] * NBUF
        for b in range(NBUF):
            j = NBUF * g + b

            @pl.when(g > 0)
            def _(b=b):
                _wait_outs(b)

            gh[b] = pltpu.async_copy(
                table_hbm.at[idx_v.at[pl.ds(j * BCHUNK * HIST, BCHUNK * HIST)]],
                stage.at[b],
                gsems.at[b])
        for b in range(NBUF):
            j = NBUF * g + b
            gh[b].wait()
            for r in range(BCHUNK):
                pltpu.async_copy(
                    stage.at[b, pl.ds(r * HIST, HIST)],
                    out_hbm.at[brow0 + j * BCHUNK + r,
                               pl.ds(0, HIST), pl.ds(0, EMBED_DIM)],
                    osems.at[b])
        return carry

    lax.fori_loop(0, NUM_CHUNKS // NBUF, group, 0, unroll=False)
    for b in range(NBUF):
        _wait_outs(b)


def kernel(x, text_embedding_vectors):
    y = _gather_kernel(text_embedding_vectors, x.reshape(-1))
    return y[:, :HIST, :EMBED_DIM]
